# baseline (speedup 1.0000x reference)
import jax
import jax.numpy as jnp
from jax.experimental import pallas as pl


def _copy_block(z_ref, o_ref):
    o_ref[...] = z_ref[...]


def kernel(z_base_global, domain_ids):
    del domain_ids
    rows, cols = z_base_global.shape
    block_rows = 1024
    grid = (rows // block_rows,)
    return pl.pallas_call(
        _copy_block,
        grid=grid,
        in_specs=[pl.BlockSpec((block_rows, cols), lambda i: (i, 0))],
        out_specs=pl.BlockSpec((block_rows, cols), lambda i: (i, 0)),
        out_shape=jax.ShapeDtypeStruct((rows, cols), z_base_global.dtype),
    )(z_base_global)
